# baseline (speedup 1.0000x reference)
import jax
import jax.numpy as jnp
from jax import lax
from jax.experimental import pallas as pl
from jax.experimental.pallas import tpu as pltpu

BLANK_ID = 0
NUM_CLASSES = 1000
BATCH = 1024
TIME = 50
BB = 16
GRID = BATCH // BB


def _one_hot_body(ids_ref, out_ref):
    iota = lax.broadcasted_iota(jnp.int32, (BB, TIME, NUM_CLASSES), 2)
    ids = ids_ref[...]
    sel = jnp.where(ids == BLANK_ID, -1, ids)[:, :, None]
    out_ref[...] = (iota == sel).astype(jnp.float32)


def kernel(outputs, outputs_length):
    ids = outputs.astype(jnp.int32)
    out = pl.pallas_call(
        _one_hot_body,
        grid=(GRID,),
        in_specs=[pl.BlockSpec((BB, TIME), lambda i: (i, 0))],
        out_specs=pl.BlockSpec((BB, TIME, NUM_CLASSES), lambda i: (i, 0, 0)),
        out_shape=jax.ShapeDtypeStruct((BATCH, TIME, NUM_CLASSES), jnp.float32),
        compiler_params=pltpu.CompilerParams(
            dimension_semantics=("parallel",),
        ),
    )(ids)
    return out, outputs_length
